# baseline (speedup 1.0000x reference)
import functools

import jax
import jax.numpy as jnp
import numpy as np
from jax import lax
from jax.experimental import pallas as pl
from jax.experimental.pallas import tpu as pltpu
from jax.experimental.pallas import tpu_sc as plsc

N = 10000
E = 320000
D = 128
EDIM = 16
K = 8
S = 32
NMP = 2
NCLS = 16
B = 4096

NUM_WORKERS = 32
CHUNK = B // NUM_WORKERS
NSLOT = CHUNK * K

_SEL_CACHE = []


def _sel_constants():
    if _SEL_CACHE:
        return _SEL_CACHE[0]
    skey = jax.random.key(42)
    sels = []
    for mp in range(NMP):
        kmp = jax.random.fold_in(skey, mp)
        sel = jax.random.randint(kmp, (N, K), 0, S).astype(jnp.int32)
        sels.append(jnp.concatenate([sel, sel], axis=1))
    sels = [jnp.pad(s.reshape(N // 8, 128), ((0, 30), (0, 0))) for s in sels]
    if not any(isinstance(s, jax.core.Tracer) for s in sels):
        _SEL_CACHE.append([np.asarray(s) for s in sels])
    return sels


try:
    _sel_constants()
except Exception:
    _SEL_CACHE.clear()


def _sc_gather_body(tid_hbm, feats_hbm, n2e0_hbm, n2e1_hbm, sel0_hbm, sel1_hbm,
                    ee0_hbm, ee1_hbm, tf_hbm, em0_hbm, em1_hbm,
                    tid_v, rows_v, n2e_v, sel_v, sidx_v, base_v,
                    fidx0_v, fidx1_v, ev0_v, ev1_v, em_v,
                    semf, semt, semg):
    wid = lax.axis_index("s") * 2 + lax.axis_index("c")
    base = wid * CHUNK

    pltpu.sync_copy(tid_hbm.at[pl.ds(base, CHUNK)], tid_v)

    cp_feats = pltpu.async_copy(feats_hbm.at[tid_v], rows_v, semf)

    lanes = lax.iota(jnp.int32, 16)
    first8 = lanes < 8
    doff = (lanes >> 3) * (E * 8) + (lanes & 7) * 128

    for c in range(CHUNK // 16):
        sidx_v[pl.ds(c * 16, 16)] = tid_v[pl.ds(c * 16, 16)] >> 3

    def build_indices(n2e_hbm, sel_hbm, fidx_v):
        c1 = pltpu.async_copy(n2e_hbm.at[tid_v], n2e_v, semt)
        c2 = pltpu.async_copy(sel_hbm.at[sidx_v], sel_v, semt)
        c1.wait()
        c2.wait()

        def idx_body(c, carry):
            hvec = (tid_v[pl.ds(c * 16, 16)] & 7) * 16
            for j in range(16):
                b = c * 16 + j
                selv = plsc.load_gather(sel_v.at[b], [hvec[j] + lanes])
                eidx = plsc.load_gather(n2e_v.at[b], [selv])
                b16 = ((eidx >> 7) << 10) | (eidx & 127)
                plsc.store_compressed(base_v.at[pl.ds(b * 8, 16)], b16, mask=first8)
            return carry

        lax.fori_loop(0, CHUNK // 16, idx_body, jnp.int32(0))

        def fidx_body(c, carry):
            bvec = base_v[pl.ds(c * 16, 16)]
            for j in range(16):
                fidx_v[pl.ds((c * 16 + j) * 16, 16)] = bvec[j] + doff
            return carry

        lax.fori_loop(0, NSLOT // 16, fidx_body, jnp.int32(0))

    def fire_gathers(ee_hbm, fidx_v, ev_v, sem):
        return [
            pltpu.async_copy(
                ee_hbm.at[fidx_v.at[pl.ds(i * 128, 128)]],
                ev_v.at[pl.ds(i * 128, 128)], sem)
            for i in range(NSLOT * 16 // 128)
        ]

    def mean_and_store(ev_v, em_hbm):
        def mean_body(b, carry):
            acc = ev_v[pl.ds(b * 128, 16)]
            for j in range(1, K):
                acc = acc + ev_v[pl.ds(b * 128 + j * 16, 16)]
            em_v[b, pl.ds(0, 16)] = acc * jnp.float32(1.0 / K)
            return carry

        lax.fori_loop(0, CHUNK, mean_body, jnp.int32(0))
        pltpu.sync_copy(em_v, em_hbm.at[pl.ds(base, CHUNK)])

    build_indices(n2e0_hbm, sel0_hbm, fidx0_v)
    copies0 = fire_gathers(ee0_hbm, fidx0_v, ev0_v, semg)
    cp_feats.wait()
    pltpu.sync_copy(rows_v, tf_hbm.at[pl.ds(base, CHUNK)])
    build_indices(n2e1_hbm, sel1_hbm, fidx1_v)
    copies1 = fire_gathers(ee1_hbm, fidx1_v, ev1_v, semf)
    for cp in copies0:
        cp.wait()
    mean_and_store(ev0_v, em0_hbm)
    for cp in copies1:
        cp.wait()
    mean_and_store(ev1_v, em1_hbm)


@functools.partial(
    pl.kernel,
    mesh=plsc.VectorSubcoreMesh(core_axis_name="c", subcore_axis_name="s"),
    compiler_params=pltpu.CompilerParams(
        needs_layout_passes=False, use_tc_tiling_on_sc=False),
    out_type=[
        jax.ShapeDtypeStruct((B, D), jnp.float32),
        jax.ShapeDtypeStruct((B, D), jnp.float32),
        jax.ShapeDtypeStruct((B, D), jnp.float32),
    ],
    scratch_types=[
        pltpu.VMEM((CHUNK,), jnp.int32),
        pltpu.VMEM((CHUNK, D), jnp.float32),
        pltpu.VMEM((CHUNK, S), jnp.int32),
        pltpu.VMEM((CHUNK, 128), jnp.int32),
        pltpu.VMEM((CHUNK,), jnp.int32),
        pltpu.VMEM((NSLOT + 8,), jnp.int32),
        pltpu.VMEM((NSLOT * 16,), jnp.int32),
        pltpu.VMEM((NSLOT * 16,), jnp.int32),
        pltpu.VMEM((NSLOT * 16,), jnp.float32),
        pltpu.VMEM((NSLOT * 16,), jnp.float32),
        pltpu.VMEM((CHUNK, D), jnp.float32),
        pltpu.SemaphoreType.DMA,
        pltpu.SemaphoreType.DMA,
        pltpu.SemaphoreType.DMA,
    ],
)
def _sc_gather(*refs):
    _sc_gather_body(*refs)


BB = 1024


def _tc_body(tf_ref, em0_ref, em1_ref, wp_ref, wep0_ref, wep1_ref,
             wna0_ref, wna1_ref, attn_ref, wfc_ref, bfc_ref,
             logits_ref, w0_ref, w1_ref):
    f32 = jnp.float32
    tf = tf_ref[...]
    dfe = jnp.dot(tf, wp_ref[...], preferred_element_type=f32)

    def head(em_ref, wep_ref, wna_ref):
        p = jnp.dot(em_ref[:, :EDIM], wep_ref[...], preferred_element_type=f32)
        h = jnp.dot(dfe, wna_ref[:D, :], preferred_element_type=f32)
        h = h + jnp.dot(p, wna_ref[D:, :], preferred_element_type=f32)
        return jnp.maximum(h, 0.0)

    h0 = head(em0_ref, wep0_ref, wna0_ref)
    h1 = head(em1_ref, wep1_ref, wna1_ref)
    s0 = jnp.dot(h0, attn_ref[...], preferred_element_type=f32)
    s1 = jnp.dot(h1, attn_ref[...], preferred_element_type=f32)
    m = jnp.maximum(s0, s1)
    e0 = jnp.exp(s0 - m)
    e1 = jnp.exp(s1 - m)
    z = e0 + e1
    w0 = e0 / z
    w1 = e1 / z
    agg = w0 * h0 + w1 * h1
    logits_ref[...] = jnp.dot(agg, wfc_ref[...], preferred_element_type=f32) + bfc_ref[...]
    w0_ref[...] = w0[:, 0]
    w1_ref[...] = w1[:, 0]


def _tc_dense(tf, em0, em1, W_prep0, Wep0, Wep1, Wna0, Wna1, attn_col, W_fc, b_fc_row):
    grid = (B // BB,)
    row_blk = lambda w: pl.BlockSpec((BB, w), lambda i: (i, 0))
    full = lambda a, b: pl.BlockSpec((a, b), lambda i: (0, 0))
    return pl.pallas_call(
        _tc_body,
        grid=grid,
        in_specs=[
            row_blk(D), row_blk(D), row_blk(D),
            full(D, D), full(EDIM, D), full(EDIM, D),
            full(2 * D, D), full(2 * D, D),
            full(D, 1), full(D, NCLS), full(1, NCLS),
        ],
        out_specs=[
            pl.BlockSpec((BB, NCLS), lambda i: (i, 0)),
            pl.BlockSpec((BB,), lambda i: (i,)),
            pl.BlockSpec((BB,), lambda i: (i,)),
        ],
        out_shape=[
            jax.ShapeDtypeStruct((B, NCLS), jnp.float32),
            jax.ShapeDtypeStruct((B,), jnp.float32),
            jax.ShapeDtypeStruct((B,), jnp.float32),
        ],
    )(tf, em0, em1, W_prep0, Wep0, Wep1, Wna0, Wna1, attn_col, W_fc, b_fc_row)


def _native_flat_view(edge_emb):
    return edge_emb.reshape(E // 128, 128, 2, 8).transpose(2, 0, 3, 1).reshape(E * EDIM)


def kernel(train_ids, feats, node2edge_idx_0, node2edge_idx_1, edge_emb_0,
           edge_emb_1, edge_node_adj_0, edge_node_adj_1, W_prep0, W_prep1,
           W_edge_prep_0, W_edge_prep_1, W_edge_agg_0, W_edge_agg_1,
           W_node_agg_0, W_node_agg_1, attn_vec, W_fc, b_fc):
    sel0, sel1 = _sel_constants()
    tf, em0, em1 = _sc_gather(
        train_ids.astype(jnp.int32), feats,
        node2edge_idx_0.astype(jnp.int32), node2edge_idx_1.astype(jnp.int32),
        jnp.asarray(sel0), jnp.asarray(sel1),
        _native_flat_view(edge_emb_0), _native_flat_view(edge_emb_1),
    )
    logits, w0, w1 = _tc_dense(
        tf, em0, em1, W_prep0, W_edge_prep_0, W_edge_prep_1,
        W_node_agg_0, W_node_agg_1,
        attn_vec.reshape(D, 1), W_fc, b_fc.reshape(1, NCLS),
    )
    return (logits, jnp.stack([w0, w1], axis=0))


# speedup vs baseline: 1.0537x; 1.0537x over previous
import functools

import jax
import jax.numpy as jnp
import numpy as np
from jax import lax
from jax.experimental import pallas as pl
from jax.experimental.pallas import tpu as pltpu
from jax.experimental.pallas import tpu_sc as plsc

N = 10000
E = 320000
D = 128
EDIM = 16
K = 8
S = 32
NMP = 2
NCLS = 16
B = 4096

NUM_WORKERS = 32
CHUNK = B // NUM_WORKERS
NSLOT = CHUNK * K

_SEL_CACHE = []


def _sel_constants():
    if _SEL_CACHE:
        return _SEL_CACHE[0]
    skey = jax.random.key(42)
    sels = []
    for mp in range(NMP):
        kmp = jax.random.fold_in(skey, mp)
        sel = jax.random.randint(kmp, (N, K), 0, S).astype(jnp.int32)
        sels.append(jnp.concatenate([sel, sel], axis=1))
    sels = [s.reshape(N * 16) for s in sels]
    if not any(isinstance(s, jax.core.Tracer) for s in sels):
        _SEL_CACHE.append([np.asarray(s) for s in sels])
    return sels


try:
    _sel_constants()
except Exception:
    _SEL_CACHE.clear()


def _sc_gather_body(tid_hbm, feats_hbm, n2e0_hbm, n2e1_hbm, sel0_hbm, sel1_hbm,
                    ee0_hbm, ee1_hbm, tf_hbm, em0_hbm, em1_hbm,
                    tid_v, rows_v, n2e_v, sel_v, base_v,
                    fidx0_v, fidx1_v, ev0_v, ev1_v, em_v,
                    semf, semt, semg):
    wid = lax.axis_index("s") * 2 + lax.axis_index("c")
    base = wid * CHUNK

    pltpu.sync_copy(tid_hbm.at[pl.ds(base, CHUNK)], tid_v)

    cp_feats = pltpu.async_copy(feats_hbm.at[tid_v], rows_v, semf)

    lanes = lax.iota(jnp.int32, 16)
    first8 = lanes < 8
    doff = (lanes >> 3) * (E * 8) + (lanes & 7) * 128

    def build_indices(n2e_hbm, sel_hbm, fidx_v):
        c1 = pltpu.async_copy(n2e_hbm.at[tid_v], n2e_v, semt)
        c2 = pltpu.async_copy(sel_hbm.at[tid_v], sel_v, semt)
        c1.wait()
        c2.wait()

        def idx_body(b, carry):
            selv = sel_v[b, :]
            eidx = plsc.load_gather(n2e_v.at[b], [selv])
            b16 = ((eidx >> 7) << 10) | (eidx & 127)
            plsc.store_compressed(base_v.at[pl.ds(b * 8, 16)], b16, mask=first8)
            return carry

        lax.fori_loop(0, CHUNK, idx_body, jnp.int32(0))

        def fidx_body(c, carry):
            bvec = base_v[pl.ds(c * 16, 16)]
            for j in range(16):
                fidx_v[pl.ds((c * 16 + j) * 16, 16)] = bvec[j] + doff
            return carry

        lax.fori_loop(0, NSLOT // 16, fidx_body, jnp.int32(0))

    def fire_gathers(ee_hbm, fidx_v, ev_v, sem):
        return [
            pltpu.async_copy(
                ee_hbm.at[fidx_v.at[pl.ds(i * 128, 128)]],
                ev_v.at[pl.ds(i * 128, 128)], sem)
            for i in range(NSLOT * 16 // 128)
        ]

    def mean_and_store(ev_v, em_hbm):
        def mean_body(b, carry):
            acc = ev_v[pl.ds(b * 128, 16)]
            for j in range(1, K):
                acc = acc + ev_v[pl.ds(b * 128 + j * 16, 16)]
            em_v[b, pl.ds(0, 16)] = acc * jnp.float32(1.0 / K)
            return carry

        lax.fori_loop(0, CHUNK, mean_body, jnp.int32(0))
        pltpu.sync_copy(em_v, em_hbm.at[pl.ds(base, CHUNK)])

    build_indices(n2e0_hbm, sel0_hbm, fidx0_v)
    copies0 = fire_gathers(ee0_hbm, fidx0_v, ev0_v, semg)
    cp_feats.wait()
    pltpu.sync_copy(rows_v, tf_hbm.at[pl.ds(base, CHUNK)])
    build_indices(n2e1_hbm, sel1_hbm, fidx1_v)
    copies1 = fire_gathers(ee1_hbm, fidx1_v, ev1_v, semf)
    for cp in copies0:
        cp.wait()
    mean_and_store(ev0_v, em0_hbm)
    for cp in copies1:
        cp.wait()
    mean_and_store(ev1_v, em1_hbm)


@functools.partial(
    pl.kernel,
    mesh=plsc.VectorSubcoreMesh(core_axis_name="c", subcore_axis_name="s"),
    compiler_params=pltpu.CompilerParams(
        needs_layout_passes=False, use_tc_tiling_on_sc=False),
    out_type=[
        jax.ShapeDtypeStruct((B, D), jnp.float32),
        jax.ShapeDtypeStruct((B, D), jnp.float32),
        jax.ShapeDtypeStruct((B, D), jnp.float32),
    ],
    scratch_types=[
        pltpu.VMEM((CHUNK,), jnp.int32),
        pltpu.VMEM((CHUNK, D), jnp.float32),
        pltpu.VMEM((CHUNK, S), jnp.int32),
        pltpu.VMEM((CHUNK, 16), jnp.int32),
        pltpu.VMEM((NSLOT + 8,), jnp.int32),
        pltpu.VMEM((NSLOT * 16,), jnp.int32),
        pltpu.VMEM((NSLOT * 16,), jnp.int32),
        pltpu.VMEM((NSLOT * 16,), jnp.float32),
        pltpu.VMEM((NSLOT * 16,), jnp.float32),
        pltpu.VMEM((CHUNK, D), jnp.float32),
        pltpu.SemaphoreType.DMA,
        pltpu.SemaphoreType.DMA,
        pltpu.SemaphoreType.DMA,
    ],
)
def _sc_gather(*refs):
    _sc_gather_body(*refs)


BB = 1024


def _tc_body(tf_ref, em0_ref, em1_ref, wp_ref, wep0_ref, wep1_ref,
             wna0_ref, wna1_ref, attn_ref, wfc_ref, bfc_ref,
             logits_ref, w0_ref, w1_ref):
    f32 = jnp.float32
    tf = tf_ref[...]
    dfe = jnp.dot(tf, wp_ref[...], preferred_element_type=f32)

    def head(em_ref, wep_ref, wna_ref):
        p = jnp.dot(em_ref[:, :EDIM], wep_ref[...], preferred_element_type=f32)
        h = jnp.dot(dfe, wna_ref[:D, :], preferred_element_type=f32)
        h = h + jnp.dot(p, wna_ref[D:, :], preferred_element_type=f32)
        return jnp.maximum(h, 0.0)

    h0 = head(em0_ref, wep0_ref, wna0_ref)
    h1 = head(em1_ref, wep1_ref, wna1_ref)
    s0 = jnp.dot(h0, attn_ref[...], preferred_element_type=f32)
    s1 = jnp.dot(h1, attn_ref[...], preferred_element_type=f32)
    m = jnp.maximum(s0, s1)
    e0 = jnp.exp(s0 - m)
    e1 = jnp.exp(s1 - m)
    z = e0 + e1
    w0 = e0 / z
    w1 = e1 / z
    agg = w0 * h0 + w1 * h1
    logits_ref[...] = jnp.dot(agg, wfc_ref[...], preferred_element_type=f32) + bfc_ref[...]
    w0_ref[...] = w0[:, 0]
    w1_ref[...] = w1[:, 0]


def _tc_dense(tf, em0, em1, W_prep0, Wep0, Wep1, Wna0, Wna1, attn_col, W_fc, b_fc_row):
    grid = (B // BB,)
    row_blk = lambda w: pl.BlockSpec((BB, w), lambda i: (i, 0))
    full = lambda a, b: pl.BlockSpec((a, b), lambda i: (0, 0))
    return pl.pallas_call(
        _tc_body,
        grid=grid,
        in_specs=[
            row_blk(D), row_blk(D), row_blk(D),
            full(D, D), full(EDIM, D), full(EDIM, D),
            full(2 * D, D), full(2 * D, D),
            full(D, 1), full(D, NCLS), full(1, NCLS),
        ],
        out_specs=[
            pl.BlockSpec((BB, NCLS), lambda i: (i, 0)),
            pl.BlockSpec((BB,), lambda i: (i,)),
            pl.BlockSpec((BB,), lambda i: (i,)),
        ],
        out_shape=[
            jax.ShapeDtypeStruct((B, NCLS), jnp.float32),
            jax.ShapeDtypeStruct((B,), jnp.float32),
            jax.ShapeDtypeStruct((B,), jnp.float32),
        ],
    )(tf, em0, em1, W_prep0, Wep0, Wep1, Wna0, Wna1, attn_col, W_fc, b_fc_row)


def _native_flat_view(edge_emb):
    return edge_emb.reshape(E // 128, 128, 2, 8).transpose(2, 0, 3, 1).reshape(E * EDIM)


def kernel(train_ids, feats, node2edge_idx_0, node2edge_idx_1, edge_emb_0,
           edge_emb_1, edge_node_adj_0, edge_node_adj_1, W_prep0, W_prep1,
           W_edge_prep_0, W_edge_prep_1, W_edge_agg_0, W_edge_agg_1,
           W_node_agg_0, W_node_agg_1, attn_vec, W_fc, b_fc):
    sel0, sel1 = _sel_constants()
    tf, em0, em1 = _sc_gather(
        train_ids.astype(jnp.int32), feats,
        node2edge_idx_0.astype(jnp.int32), node2edge_idx_1.astype(jnp.int32),
        jnp.asarray(sel0).reshape(N, 16), jnp.asarray(sel1).reshape(N, 16),
        _native_flat_view(edge_emb_0), _native_flat_view(edge_emb_1),
    )
    logits, w0, w1 = _tc_dense(
        tf, em0, em1, W_prep0, W_edge_prep_0, W_edge_prep_1,
        W_node_agg_0, W_node_agg_1,
        attn_vec.reshape(D, 1), W_fc, b_fc.reshape(1, NCLS),
    )
    return (logits, jnp.stack([w0, w1], axis=0))
